# baseline (speedup 1.0000x reference)
import functools

import jax
import jax.numpy as jnp
from jax.experimental import pallas as pl
from jax.experimental.pallas import tpu as pltpu


def _pos_body(row_ref, col_ref, depth_ref, out_ref, *, hb, w):
    di = pl.program_id(0)
    col = col_ref[0:w, :]
    row = row_ref[...]
    depth = depth_ref[pl.ds(di, 1), :]
    zc = jnp.zeros((w, 64), jnp.float32)
    zr = jnp.zeros((hb, 64), jnp.float32)
    x = jnp.concatenate(
        [col, zc, jnp.broadcast_to(depth, (w, 64))], axis=-1)
    y = jnp.concatenate([zr, row, zr], axis=-1)
    out_ref[...] = (x[None, :, :] + y[:, None, :])[None]


def kernel(scan, row_weight, col_weight, depth_weight):
    d, em, h, w = scan.shape
    hb = 32
    n_h = h // hb
    body = functools.partial(_pos_body, hb=hb, w=w)
    out = pl.pallas_call(
        body,
        grid=(d, n_h),
        in_specs=[
            pl.BlockSpec((hb, 64), lambda di, hi: (hi, 0)),
            pl.BlockSpec((256, 64), lambda di, hi: (0, 0)),
            pl.BlockSpec((40, 64), lambda di, hi: (0, 0)),
        ],
        out_specs=pl.BlockSpec((1, hb, w, 192), lambda di, hi: (di, hi, 0, 0)),
        out_shape=jax.ShapeDtypeStruct((d, h, w, 192), jnp.float32),
        compiler_params=pltpu.CompilerParams(
            dimension_semantics=("parallel", "parallel")),
    )(row_weight, col_weight, depth_weight)
    return out
